# baseline (speedup 1.0000x reference)
import functools

import numpy as np
import jax
import jax.numpy as jnp
from jax import lax
from jax.experimental import pallas as pl
from jax.experimental.pallas import tpu as pltpu
from jax.experimental.pallas import tpu_sc as plsc

_VOCAB = 100000
_EMBED = 1024
_MAX_LEN = 2048
_B = 4
_S = 2048

_NW = 32
_SPW = _S // _NW
_C = 16
_SUBS = _SPW // _C
_NCH = _B * _SUBS
_RR = 3
_RP = 2
_HG = _EMBED // 32


def _make_pe_packed():
    pos = np.arange(_MAX_LEN, dtype=np.float32)[:, None]
    div = np.exp(np.arange(0, _EMBED, 2, dtype=np.float32)
                 * -(np.log(10000.0) / _EMBED))
    pe = np.zeros((_MAX_LEN, _EMBED), np.float32)
    pe[:, 0::2] = np.sin(pos * div)
    pe[:, 1::2] = np.cos(pos * div)
    pe = pe[:_S]
    bf = pe.astype(jnp.bfloat16).view(np.uint16)
    lo = bf[:, :_EMBED // 2].astype(np.uint32)
    hi = bf[:, _EMBED // 2:].astype(np.uint32)
    return lo | (hi << 16)


_PE_PACKED = _make_pe_packed()

_mesh = plsc.VectorSubcoreMesh(core_axis_name="c", subcore_axis_name="s")


@functools.partial(
    pl.kernel,
    out_type=jax.ShapeDtypeStruct((_B * _S, _EMBED), jnp.float32),
    mesh=_mesh,
    scratch_types=[
        pltpu.VMEM((_B, _SPW), jnp.int32),
        pltpu.VMEM((_SPW, _EMBED // 2), jnp.uint32),
        pltpu.VMEM((_RR, _C, _EMBED), jnp.float32),
        pltpu.VMEM((_RP, _C, _EMBED), jnp.float32),
        pltpu.SemaphoreType.DMA,
        pltpu.SemaphoreType.DMA,
        pltpu.SemaphoreType.DMA,
        pltpu.SemaphoreType.DMA,
        pltpu.SemaphoreType.DMA,
        pltpu.SemaphoreType.DMA,
        pltpu.SemaphoreType.DMA,
        pltpu.SemaphoreType.DMA,
    ],
)
def _emb_kernel(seq_hbm, param_hbm, pe_hbm, table_hbm, out_hbm,
                idx_v, pe_v, rows_v, par_v, *sems):
    sema = sems[0:_RR]
    semp = sems[_RR:_RR + _RP]
    semo = sems[_RR + _RP:2 * _RR + _RP]
    cid = lax.axis_index("c")
    sid = lax.axis_index("s")
    wid = sid * 2 + cid
    s_base = wid * _SPW

    def _roff(c):
        return s_base + (c // _SUBS) * _S + (c % _SUBS) * _C

    def _idx(c):
        return idx_v.at[c // _SUBS, pl.ds((c % _SUBS) * _C, _C)]

    def issue_in(c):
        br, bp = c % _RR, c % _RP
        pltpu.async_copy(table_hbm.at[_idx(c)], rows_v.at[br], sema[br])
        pltpu.async_copy(param_hbm.at[pl.ds(_roff(c), _C)], par_v.at[bp],
                         semp[bp])

    def wait_in(c):
        br, bp = c % _RR, c % _RP
        pltpu.make_async_copy(table_hbm.at[_idx(c)], rows_v.at[br],
                              sema[br]).wait()
        pltpu.make_async_copy(param_hbm.at[pl.ds(_roff(c), _C)],
                              par_v.at[bp], semp[bp]).wait()

    def issue_store(c):
        br = c % _RR
        pltpu.async_copy(rows_v.at[br], out_hbm.at[pl.ds(_roff(c), _C)],
                         semo[br])

    def wait_store(c):
        br = c % _RR
        pltpu.make_async_copy(rows_v.at[br], out_hbm.at[pl.ds(_roff(c), _C)],
                              semo[br]).wait()

    _MASK = jnp.uint32(0xFFFF0000)

    def add_pass(c):
        br, bp = c % _RR, c % _RP
        prow = (c % _SUBS) * _C

        @plsc.parallel_loop(0, _C * _HG, 1, unroll=4)
        def addgrp(i):
            r = i // _HG
            j = i - r * _HG
            pk = pe_v[prow + r, pl.ds(j * 16, 16)]
            pe_lo = lax.bitcast_convert_type(pk << 16, jnp.float32)
            pe_hi = lax.bitcast_convert_type(pk & _MASK, jnp.float32)
            sl_lo = pl.ds(j * 16, 16)
            sl_hi = pl.ds(_EMBED // 2 + j * 16, 16)
            plsc.addupdate(rows_v.at[br, r, sl_lo],
                           par_v[bp, r, sl_lo] + pe_lo)
            plsc.addupdate(rows_v.at[br, r, sl_hi],
                           par_v[bp, r, sl_hi] + pe_hi)

    for b in range(_B):
        pltpu.sync_copy(seq_hbm.at[pl.ds(b * _S + s_base, _SPW)],
                        idx_v.at[b])
    issue_in(0)
    pltpu.sync_copy(pe_hbm.at[pl.ds(s_base, _SPW)], pe_v)
    for c in range(_NCH):
        if c + 1 < _NCH:
            if c >= 2:
                wait_store(c - 2)
            issue_in(c + 1)
        wait_in(c)
        add_pass(c)
        issue_store(c)
    wait_store(_NCH - 2)
    wait_store(_NCH - 1)


def kernel(sequence, param_embedding, token_table):
    seq = sequence.astype(jnp.int32).reshape(_B * _S)
    param = param_embedding.reshape(_B * _S, _EMBED)
    pe = jnp.asarray(_PE_PACKED)
    out = _emb_kernel(seq, param, pe, token_table)
    return out.reshape(_B, _S, _EMBED)


# speedup vs baseline: 1.0108x; 1.0108x over previous
import functools

import numpy as np
import jax
import jax.numpy as jnp
from jax import lax
from jax.experimental import pallas as pl
from jax.experimental.pallas import tpu as pltpu
from jax.experimental.pallas import tpu_sc as plsc

_VOCAB = 100000
_EMBED = 1024
_MAX_LEN = 2048
_B = 4
_S = 2048

_NW = 32
_SPW = _S // _NW
_C = 16
_SUBS = _SPW // _C
_NCH = _B * _SUBS
_RR = 3
_RP = 2
_HG = _EMBED // 32


def _make_pe_packed():
    pos = np.arange(_MAX_LEN, dtype=np.float32)[:, None]
    div = np.exp(np.arange(0, _EMBED, 2, dtype=np.float32)
                 * -(np.log(10000.0) / _EMBED))
    pe = np.zeros((_MAX_LEN, _EMBED), np.float32)
    pe[:, 0::2] = np.sin(pos * div)
    pe[:, 1::2] = np.cos(pos * div)
    pe = pe[:_S]
    bf = pe.astype(jnp.bfloat16).view(np.uint16)
    lo = bf[:, :_EMBED // 2].astype(np.uint32)
    hi = bf[:, _EMBED // 2:].astype(np.uint32)
    return lo | (hi << 16)


_PE_PACKED = _make_pe_packed()

_mesh = plsc.VectorSubcoreMesh(core_axis_name="c", subcore_axis_name="s")


@functools.partial(
    pl.kernel,
    out_type=jax.ShapeDtypeStruct((_B, _S, _EMBED), jnp.float32),
    mesh=_mesh,
    scratch_types=[
        pltpu.VMEM((_B, _SPW), jnp.int32),
        pltpu.VMEM((_SPW, _EMBED // 2), jnp.uint32),
        pltpu.VMEM((_RR, _C, _EMBED), jnp.float32),
        pltpu.VMEM((_RP, _C, _EMBED), jnp.float32),
        pltpu.SemaphoreType.DMA,
        pltpu.SemaphoreType.DMA,
        pltpu.SemaphoreType.DMA,
        pltpu.SemaphoreType.DMA,
        pltpu.SemaphoreType.DMA,
        pltpu.SemaphoreType.DMA,
        pltpu.SemaphoreType.DMA,
        pltpu.SemaphoreType.DMA,
    ],
)
def _emb_kernel(seq_hbm, param_hbm, pe_hbm, table_hbm, out_hbm,
                idx_v, pe_v, rows_v, par_v, *sems):
    sema = sems[0:_RR]
    semp = sems[_RR:_RR + _RP]
    semo = sems[_RR + _RP:2 * _RR + _RP]
    cid = lax.axis_index("c")
    sid = lax.axis_index("s")
    wid = sid * 2 + cid
    s_base = wid * _SPW

    def _boff(c):
        return c // _SUBS, s_base + (c % _SUBS) * _C

    def _idx(c):
        return idx_v.at[c // _SUBS, pl.ds((c % _SUBS) * _C, _C)]

    def issue_in(c):
        br, bp = c % _RR, c % _RP
        pltpu.async_copy(table_hbm.at[_idx(c)], rows_v.at[br], sema[br])
        b, off = _boff(c)
        pltpu.async_copy(param_hbm.at[b, pl.ds(off, _C)], par_v.at[bp],
                         semp[bp])

    def wait_in(c):
        br, bp = c % _RR, c % _RP
        pltpu.make_async_copy(table_hbm.at[_idx(c)], rows_v.at[br],
                              sema[br]).wait()
        b, off = _boff(c)
        pltpu.make_async_copy(param_hbm.at[b, pl.ds(off, _C)],
                              par_v.at[bp], semp[bp]).wait()

    def issue_store(c):
        br = c % _RR
        b, off = _boff(c)
        pltpu.async_copy(rows_v.at[br], out_hbm.at[b, pl.ds(off, _C)],
                         semo[br])

    def wait_store(c):
        br = c % _RR
        b, off = _boff(c)
        pltpu.make_async_copy(rows_v.at[br], out_hbm.at[b, pl.ds(off, _C)],
                              semo[br]).wait()

    _MASK = jnp.uint32(0xFFFF0000)

    def add_pass(c):
        br, bp = c % _RR, c % _RP
        prow = (c % _SUBS) * _C

        @plsc.parallel_loop(0, _C * _HG, 1, unroll=4)
        def addgrp(i):
            r = i // _HG
            j = i - r * _HG
            pk = pe_v[prow + r, pl.ds(j * 16, 16)]
            pe_lo = lax.bitcast_convert_type(pk << 16, jnp.float32)
            pe_hi = lax.bitcast_convert_type(pk & _MASK, jnp.float32)
            sl_lo = pl.ds(j * 16, 16)
            sl_hi = pl.ds(_EMBED // 2 + j * 16, 16)
            plsc.addupdate(rows_v.at[br, r, sl_lo],
                           par_v[bp, r, sl_lo] + pe_lo)
            plsc.addupdate(rows_v.at[br, r, sl_hi],
                           par_v[bp, r, sl_hi] + pe_hi)

    for b in range(_B):
        pltpu.sync_copy(seq_hbm.at[b, pl.ds(s_base, _SPW)], idx_v.at[b])
    issue_in(0)
    pltpu.sync_copy(pe_hbm.at[pl.ds(s_base, _SPW)], pe_v)
    for c in range(_NCH):
        if c + 1 < _NCH:
            if c >= 2:
                wait_store(c - 2)
            issue_in(c + 1)
        wait_in(c)
        add_pass(c)
        issue_store(c)
    wait_store(_NCH - 2)
    wait_store(_NCH - 1)


def kernel(sequence, param_embedding, token_table):
    seq = sequence.astype(jnp.int32)
    pe = jnp.asarray(_PE_PACKED)
    return _emb_kernel(seq, param_embedding, pe, token_table)
